# baseline (speedup 1.0000x reference)
import functools

import jax
import jax.numpy as jnp
from jax.experimental import pallas as pl
from jax.experimental.pallas import tpu as pltpu

V, FIN, H = 512, 32, 64
TI = 32
NT = V // TI
PA, P2, PB, P3 = NT, NT + 1, 2 * NT + 1, 2 * NT + 2
STEPS = 3 * NT + 2
HIGH = jax.lax.Precision.HIGHEST


def _e1t(row0, sv, a1_scr, bc0t_scr, cm_scr):
    a1t = a1_scr[pl.ds(row0, TI), :]
    cdt = cm_scr[:, 0:1]
    return (a1t[:, :, None] + bc0t_scr[...][None, :, :]
            + sv[:, None, :] * cdt[None, :, :])


def _s1t(row0, sv, a1_scr, bc0t_scr, cm_scr, st1t_scr):
    e1t = _e1t(row0, sv, a1_scr, bc0t_scr, cm_scr)
    mu1t = st1t_scr[:, 0:1][None, :, :]
    inv1t = st1t_scr[:, 1:2][None, :, :]
    return jax.nn.relu((e1t - mu1t) * inv1t)


def _norm_relu(h):
    mu = jnp.mean(h, axis=0, keepdims=True)
    var = jnp.mean((h - mu) ** 2, axis=0, keepdims=True)
    return jax.nn.relu((h - mu) / jnp.sqrt(var + 1e-5))


def _mega(x_ref, s_ref, a_ref, emb_ref,
          wa1_ref, wb1_ref, wc1_ref, wv1_ref, wu1_ref,
          wa2_ref, wb2_ref, wv2_ref, wu2_ref, wc2_ref,
          x2_ref, s2_ref,
          num_scr, den_scr, raw1_scr, raw2_scr,
          x1_scr, a2_scr, u2_scr, e1s_scr,
          a1_scr, bc0_scr, bc0t_scr, v1t_scr, b2t_scr, v2t_scr,
          cm_scr, st1t_scr, st2t_scr, wc2t_scr):
    i = pl.program_id(0)

    @pl.when(i == 0)
    def _init():
        xf = x_ref[...]
        a1_scr[...] = jnp.dot(xf, wa1_ref[...], precision=HIGH)
        c = jnp.dot(emb_ref[...], wc1_ref[...], precision=HIGH)
        c0 = c[0:1, :]
        cd = c[1:2, :] - c0
        bc0 = jnp.dot(xf, wb1_ref[...], precision=HIGH) + c0
        bc0_scr[...] = bc0
        bc0t_scr[...] = jnp.transpose(bc0)
        v1t_scr[...] = jnp.transpose(
            jnp.dot(xf, wv1_ref[...], precision=HIGH))
        cm_scr[...] = jnp.concatenate(
            [jnp.transpose(cd), jnp.zeros((H, 7), jnp.float32)], axis=1)
        wc2t_scr[...] = jnp.transpose(wc2_ref[...])

    @pl.when(i < PA)
    def _phase1():
        li = i
        row0 = li * TI
        sv = s_ref[...].astype(jnp.float32)
        e1t = _e1t(row0, sv, a1_scr, bc0t_scr, cm_scr)
        e1s_scr[pl.ds(row0, TI), :, :] = e1t.astype(jnp.bfloat16)
        adj = jnp.minimum(a_ref[...] + s_ref[...], 1).astype(jnp.float32)
        g = jax.nn.sigmoid(e1t)
        ga = g * adj[:, None, :]
        t2 = ga * v1t_scr[...][None, :, :]
        onesv = jnp.ones((V,), jnp.float32)
        num_scr[pl.ds(row0, TI), :] = jax.lax.dot_general(
            t2, onesv, (((2,), (0,)), ((), ())), precision=HIGH)
        den_scr[pl.ds(row0, TI), :] = jax.lax.dot_general(
            ga, onesv, (((2,), (0,)), ((), ())), precision=HIGH)
        rt = jnp.sum(sv, axis=1, keepdims=True)
        qt = jnp.sum(sv, axis=0, keepdims=True)
        ca = jnp.sum(a1_scr[pl.ds(row0, TI), :] * rt, axis=0, keepdims=True)
        cb = jnp.transpose(
            jnp.sum(bc0t_scr[...] * qt, axis=1, keepdims=True))
        n1 = jnp.broadcast_to(jnp.sum(rt, axis=0, keepdims=True), (1, H))
        upd = jnp.concatenate(
            [ca, cb, n1, jnp.zeros((5, H), jnp.float32)], axis=0)

        @pl.when(li == 0)
        def _():
            raw1_scr[...] = upd

        @pl.when(li > 0)
        def _():
            raw1_scr[...] = raw1_scr[...] + upd

    @pl.when(i == PA)
    def _prep_a():
        xf = x_ref[...]
        ux1 = jnp.dot(xf, wu1_ref[...], precision=HIGH)
        x1 = _norm_relu(ux1 + num_scr[...] / (den_scr[...] + 1e-6))
        x1_scr[...] = x1
        a2_scr[...] = jnp.dot(x1, wa2_ref[...], precision=HIGH)
        b2t_scr[...] = jnp.transpose(
            jnp.dot(x1, wb2_ref[...], precision=HIGH))
        v2t_scr[...] = jnp.transpose(
            jnp.dot(x1, wv2_ref[...], precision=HIGH))
        u2_scr[...] = jnp.dot(x1, wu2_ref[...], precision=HIGH)
        a1 = a1_scr[...]
        bc0 = bc0_scr[...]
        cd = jnp.transpose(cm_scr[:, 0:1])
        sa = jnp.sum(a1, axis=0, keepdims=True)
        sa2 = jnp.sum(a1 * a1, axis=0, keepdims=True)
        sb = jnp.sum(bc0, axis=0, keepdims=True)
        sb2 = jnp.sum(bc0 * bc0, axis=0, keepdims=True)
        ca = raw1_scr[0:1, :]
        cb = raw1_scr[1:2, :]
        n1 = raw1_scr[2:3, :]
        fV = jnp.float32(V)
        n = fV * fV
        se = fV * sa + fV * sb + n1 * cd
        se2 = (fV * sa2 + fV * sb2 + 2.0 * sa * sb
               + 2.0 * cd * (ca + cb) + cd * cd * n1)
        mu1 = se / n
        var1 = se2 / n - mu1 * mu1
        inv1 = jax.lax.rsqrt(var1 + 1e-5)
        st1t_scr[...] = jnp.concatenate(
            [jnp.transpose(mu1), jnp.transpose(inv1),
             jnp.zeros((H, 6), jnp.float32)], axis=1)

    @pl.when((i > PA) & (i < PB))
    def _phase2():
        li = i - P2
        row0 = li * TI
        e1b = e1s_scr[pl.ds(row0, TI), :, :].astype(jnp.float32)
        mu1t = st1t_scr[:, 0:1][None, :, :]
        inv1t = st1t_scr[:, 1:2][None, :, :]
        s1t = jax.nn.relu((e1b - mu1t) * inv1t)
        e1s_scr[pl.ds(row0, TI), :, :] = s1t.astype(jnp.bfloat16)
        wc2t = jnp.broadcast_to(wc2t_scr[...][None, :, :], (TI, H, H))
        sct = jax.lax.dot_general(
            wc2t, s1t, (((2,), (1,)), ((0,), (0,))))
        a2t = a2_scr[pl.ds(row0, TI), :]
        e2t = a2t[:, :, None] + b2t_scr[...][None, :, :] + sct
        adj = jnp.minimum(a_ref[...] + s_ref[...], 1).astype(jnp.float32)
        g = jax.nn.sigmoid(e2t)
        ga = g * adj[:, None, :]
        t2 = ga * v2t_scr[...][None, :, :]
        onesv = jnp.ones((V,), jnp.float32)
        num_scr[pl.ds(row0, TI), :] = jax.lax.dot_general(
            t2, onesv, (((2,), (0,)), ((), ())), precision=HIGH)
        den_scr[pl.ds(row0, TI), :] = jax.lax.dot_general(
            ga, onesv, (((2,), (0,)), ((), ())), precision=HIGH)
        es = jnp.sum(jax.lax.dot_general(
            e2t, onesv, (((2,), (0,)), ((), ())), precision=HIGH),
            axis=0, keepdims=True)
        eq = jnp.sum(jax.lax.dot_general(
            e2t * e2t, onesv, (((2,), (0,)), ((), ())), precision=HIGH),
            axis=0, keepdims=True)
        upd = jnp.concatenate(
            [es, eq, jnp.zeros((6, H), jnp.float32)], axis=0)

        @pl.when(li == 0)
        def _():
            raw2_scr[...] = upd

        @pl.when(li > 0)
        def _():
            raw2_scr[...] = raw2_scr[...] + upd

    @pl.when(i == PB)
    def _prep_b():
        h = u2_scr[...] + num_scr[...] / (den_scr[...] + 1e-6)
        x2_ref[...] = x1_scr[...] + _norm_relu(h)
        n = jnp.float32(V) * jnp.float32(V)
        mu2 = raw2_scr[0:1, :] / n
        var2 = raw2_scr[1:2, :] / n - mu2 * mu2
        inv2 = jax.lax.rsqrt(var2 + 1e-5)
        st2t_scr[...] = jnp.concatenate(
            [jnp.transpose(mu2), jnp.transpose(inv2),
             jnp.zeros((H, 6), jnp.float32)], axis=1)

    @pl.when(i >= P3)
    def _phase3():
        li = i - P3
        row0 = li * TI
        s1t = e1s_scr[pl.ds(row0, TI), :, :].astype(jnp.float32)
        wc2t = jnp.broadcast_to(wc2t_scr[...][None, :, :], (TI, H, H))
        sct = jax.lax.dot_general(
            wc2t, s1t, (((2,), (1,)), ((0,), (0,))))
        a2t = a2_scr[pl.ds(row0, TI), :]
        e2t = a2t[:, :, None] + b2t_scr[...][None, :, :] + sct
        mu2t = st2t_scr[:, 0:1][None, :, :]
        inv2t = st2t_scr[:, 1:2][None, :, :]
        s2_ref[...] = s1t + jax.nn.relu((e2t - mu2t) * inv2t)


def _tile_idx(i):
    return jnp.where(i < P2, jnp.minimum(i, NT - 1),
                     jnp.minimum(i - P2, NT - 1))


def _full(shape):
    return pl.BlockSpec(shape, lambda i: tuple(0 for _ in shape))


@functools.partial(jax.jit, static_argnums=())
def kernel(x, A, S, emb, Wu1, Wv1, Wa1, Wb1, Wc1, Wu2, Wv2, Wa2, Wb2, Wc2):
    x2d = x[0]
    Si = S[0].astype(jnp.int32)
    Ai = A[0].astype(jnp.int32)

    f32 = jnp.float32
    rowspec = pl.BlockSpec((TI, V), lambda i: (_tile_idx(i), 0))

    x2, s2t = pl.pallas_call(
        _mega,
        compiler_params=pltpu.CompilerParams(
            vmem_limit_bytes=63 * 1024 * 1024),
        grid=(STEPS,),
        in_specs=[_full((V, FIN)), rowspec, rowspec, _full((2, FIN)),
                  _full((FIN, H)), _full((FIN, H)), _full((FIN, H)),
                  _full((FIN, H)), _full((FIN, H)),
                  _full((H, H)), _full((H, H)), _full((H, H)),
                  _full((H, H)), _full((H, H))],
        out_specs=[_full((V, H)),
                   pl.BlockSpec((TI, H, V),
                                lambda i: (jnp.clip(i - P3, 0, NT - 1), 0, 0))],
        out_shape=[jax.ShapeDtypeStruct((V, H), f32),
                   jax.ShapeDtypeStruct((V, H, V), f32)],
        scratch_shapes=[
            pltpu.VMEM((V, H), f32),
            pltpu.VMEM((V, H), f32),
            pltpu.VMEM((8, H), f32),
            pltpu.VMEM((8, H), f32),
            pltpu.VMEM((V, H), f32),
            pltpu.VMEM((V, H), f32),
            pltpu.VMEM((V, H), f32),
            pltpu.VMEM((V, H, V), jnp.bfloat16),
            pltpu.VMEM((V, H), f32),
            pltpu.VMEM((V, H), f32),
            pltpu.VMEM((H, V), f32),
            pltpu.VMEM((H, V), f32),
            pltpu.VMEM((H, V), f32),
            pltpu.VMEM((H, V), f32),
            pltpu.VMEM((H, 8), f32),
            pltpu.VMEM((H, 8), f32),
            pltpu.VMEM((H, 8), f32),
            pltpu.VMEM((H, H), f32),
        ],
    )(x2d, Si, Ai, emb, Wa1, Wb1, Wc1, Wv1, Wu1, Wa2, Wb2, Wv2, Wu2, Wc2)

    s2 = jnp.transpose(s2t, (0, 2, 1))
    return (x2[None], s2[None])


# speedup vs baseline: 1.0304x; 1.0304x over previous
import functools

import jax
import jax.numpy as jnp
from jax.experimental import pallas as pl
from jax.experimental.pallas import tpu as pltpu

V, FIN, H = 512, 32, 64
TI = 32
NT = V // TI
PA, P2, PB, P3 = NT, NT + 1, 2 * NT + 1, 2 * NT + 2
STEPS = 3 * NT + 2
HIGH = jax.lax.Precision.HIGHEST


def _e1t(row0, sv, a1_scr, bc0t_scr, cm_scr):
    a1t = a1_scr[pl.ds(row0, TI), :]
    cdt = cm_scr[:, 0:1]
    return (a1t[:, :, None] + bc0t_scr[...][None, :, :]
            + sv[:, None, :] * cdt[None, :, :])


def _s1t(row0, sv, a1_scr, bc0t_scr, cm_scr, st1t_scr):
    e1t = _e1t(row0, sv, a1_scr, bc0t_scr, cm_scr)
    mu1t = st1t_scr[:, 0:1][None, :, :]
    inv1t = st1t_scr[:, 1:2][None, :, :]
    return jax.nn.relu((e1t - mu1t) * inv1t)


def _norm_relu(h):
    mu = jnp.mean(h, axis=0, keepdims=True)
    var = jnp.mean((h - mu) ** 2, axis=0, keepdims=True)
    return jax.nn.relu((h - mu) / jnp.sqrt(var + 1e-5))


def _mega(x_ref, s_ref, a_ref, emb_ref,
          wa1_ref, wb1_ref, wc1_ref, wv1_ref, wu1_ref,
          wa2_ref, wb2_ref, wv2_ref, wu2_ref, wc2_ref,
          x2_ref, s2_ref,
          num_scr, den_scr, raw1_scr, raw2_scr,
          x1_scr, a2_scr, u2_scr, e1s_scr,
          a1_scr, bc0_scr, bc0t_scr, v1t_scr, b2t_scr, v2t_scr,
          cm_scr, st1t_scr, st2t_scr, wc2t_scr):
    i = pl.program_id(0)

    @pl.when(i == 0)
    def _init():
        xf = x_ref[...]
        a1_scr[...] = jnp.dot(xf, wa1_ref[...], precision=HIGH)
        c = jnp.dot(emb_ref[...], wc1_ref[...], precision=HIGH)
        c0 = c[0:1, :]
        cd = c[1:2, :] - c0
        bc0 = jnp.dot(xf, wb1_ref[...], precision=HIGH) + c0
        bc0_scr[...] = bc0
        bc0t_scr[...] = jnp.transpose(bc0)
        v1t_scr[...] = jnp.transpose(
            jnp.dot(xf, wv1_ref[...], precision=HIGH))
        cm_scr[...] = jnp.concatenate(
            [jnp.transpose(cd), jnp.zeros((H, 7), jnp.float32)], axis=1)
        wc2t_scr[...] = jnp.transpose(wc2_ref[...])

    @pl.when(i < PA)
    def _phase1():
        li = i
        row0 = li * TI
        sv = s_ref[...].astype(jnp.float32)
        e1t = _e1t(row0, sv, a1_scr, bc0t_scr, cm_scr)
        e1s_scr[pl.ds(row0, TI), :, :] = e1t.astype(jnp.bfloat16)
        adj = jnp.minimum(a_ref[...] + s_ref[...], 1).astype(jnp.float32)
        g = 0.5 * jnp.tanh(0.5 * e1t) + 0.5
        ga = g * adj[:, None, :]
        num_scr[pl.ds(row0, TI), :] = jnp.sum(
            ga * v1t_scr[...][None, :, :], axis=2)
        den_scr[pl.ds(row0, TI), :] = jnp.sum(ga, axis=2)
        rt = jnp.sum(sv, axis=1, keepdims=True)
        qt = jnp.sum(sv, axis=0, keepdims=True)
        ca = jnp.sum(a1_scr[pl.ds(row0, TI), :] * rt, axis=0, keepdims=True)
        cb = jnp.transpose(
            jnp.sum(bc0t_scr[...] * qt, axis=1, keepdims=True))
        n1 = jnp.broadcast_to(jnp.sum(rt, axis=0, keepdims=True), (1, H))
        upd = jnp.concatenate(
            [ca, cb, n1, jnp.zeros((5, H), jnp.float32)], axis=0)

        @pl.when(li == 0)
        def _():
            raw1_scr[...] = upd

        @pl.when(li > 0)
        def _():
            raw1_scr[...] = raw1_scr[...] + upd

    @pl.when(i == PA)
    def _prep_a():
        xf = x_ref[...]
        ux1 = jnp.dot(xf, wu1_ref[...], precision=HIGH)
        x1 = _norm_relu(ux1 + num_scr[...] / (den_scr[...] + 1e-6))
        x1_scr[...] = x1
        a2_scr[...] = jnp.dot(x1, wa2_ref[...], precision=HIGH)
        b2t_scr[...] = jnp.transpose(
            jnp.dot(x1, wb2_ref[...], precision=HIGH))
        v2t_scr[...] = jnp.transpose(
            jnp.dot(x1, wv2_ref[...], precision=HIGH))
        u2_scr[...] = jnp.dot(x1, wu2_ref[...], precision=HIGH)
        a1 = a1_scr[...]
        bc0 = bc0_scr[...]
        cd = jnp.transpose(cm_scr[:, 0:1])
        sa = jnp.sum(a1, axis=0, keepdims=True)
        sa2 = jnp.sum(a1 * a1, axis=0, keepdims=True)
        sb = jnp.sum(bc0, axis=0, keepdims=True)
        sb2 = jnp.sum(bc0 * bc0, axis=0, keepdims=True)
        ca = raw1_scr[0:1, :]
        cb = raw1_scr[1:2, :]
        n1 = raw1_scr[2:3, :]
        fV = jnp.float32(V)
        n = fV * fV
        se = fV * sa + fV * sb + n1 * cd
        se2 = (fV * sa2 + fV * sb2 + 2.0 * sa * sb
               + 2.0 * cd * (ca + cb) + cd * cd * n1)
        mu1 = se / n
        var1 = se2 / n - mu1 * mu1
        inv1 = jax.lax.rsqrt(var1 + 1e-5)
        st1t_scr[...] = jnp.concatenate(
            [jnp.transpose(mu1), jnp.transpose(inv1),
             jnp.zeros((H, 6), jnp.float32)], axis=1)

    @pl.when((i > PA) & (i < PB))
    def _phase2():
        li = i - P2
        row0 = li * TI
        e1b = e1s_scr[pl.ds(row0, TI), :, :].astype(jnp.float32)
        mu1t = st1t_scr[:, 0:1][None, :, :]
        inv1t = st1t_scr[:, 1:2][None, :, :]
        s1t = jax.nn.relu((e1b - mu1t) * inv1t)
        e1s_scr[pl.ds(row0, TI), :, :] = s1t.astype(jnp.bfloat16)
        wc2t = jnp.broadcast_to(wc2t_scr[...][None, :, :], (TI, H, H))
        sct = jax.lax.dot_general(
            wc2t, s1t, (((2,), (1,)), ((0,), (0,))))
        a2t = a2_scr[pl.ds(row0, TI), :]
        e2t = a2t[:, :, None] + b2t_scr[...][None, :, :] + sct
        adj = jnp.minimum(a_ref[...] + s_ref[...], 1).astype(jnp.float32)
        g = 0.5 * jnp.tanh(0.5 * e2t) + 0.5
        ga = g * adj[:, None, :]
        num_scr[pl.ds(row0, TI), :] = jnp.sum(
            ga * v2t_scr[...][None, :, :], axis=2)
        den_scr[pl.ds(row0, TI), :] = jnp.sum(ga, axis=2)
        es = jnp.transpose(jnp.sum(e2t, axis=(0, 2))[:, None])
        eq = jnp.transpose(jnp.sum(e2t * e2t, axis=(0, 2))[:, None])
        upd = jnp.concatenate(
            [es, eq, jnp.zeros((6, H), jnp.float32)], axis=0)

        @pl.when(li == 0)
        def _():
            raw2_scr[...] = upd

        @pl.when(li > 0)
        def _():
            raw2_scr[...] = raw2_scr[...] + upd

    @pl.when(i == PB)
    def _prep_b():
        h = u2_scr[...] + num_scr[...] / (den_scr[...] + 1e-6)
        x2_ref[...] = x1_scr[...] + _norm_relu(h)
        n = jnp.float32(V) * jnp.float32(V)
        mu2 = raw2_scr[0:1, :] / n
        var2 = raw2_scr[1:2, :] / n - mu2 * mu2
        inv2 = jax.lax.rsqrt(var2 + 1e-5)
        st2t_scr[...] = jnp.concatenate(
            [jnp.transpose(mu2), jnp.transpose(inv2),
             jnp.zeros((H, 6), jnp.float32)], axis=1)

    @pl.when(i >= P3)
    def _phase3():
        li = i - P3
        row0 = li * TI
        s1t = e1s_scr[pl.ds(row0, TI), :, :].astype(jnp.float32)
        wc2t = jnp.broadcast_to(wc2t_scr[...][None, :, :], (TI, H, H))
        sct = jax.lax.dot_general(
            wc2t, s1t, (((2,), (1,)), ((0,), (0,))))
        a2t = a2_scr[pl.ds(row0, TI), :]
        e2t = a2t[:, :, None] + b2t_scr[...][None, :, :] + sct
        mu2t = st2t_scr[:, 0:1][None, :, :]
        inv2t = st2t_scr[:, 1:2][None, :, :]
        s2_ref[...] = s1t + jax.nn.relu((e2t - mu2t) * inv2t)


def _tile_idx(i):
    return jnp.where(i < P2, jnp.minimum(i, NT - 1),
                     jnp.minimum(i - P2, NT - 1))


def _full(shape):
    return pl.BlockSpec(shape, lambda i: tuple(0 for _ in shape))


@functools.partial(jax.jit, static_argnums=())
def kernel(x, A, S, emb, Wu1, Wv1, Wa1, Wb1, Wc1, Wu2, Wv2, Wa2, Wb2, Wc2):
    x2d = x[0]
    Si = S[0].astype(jnp.int32)
    Ai = A[0].astype(jnp.int32)

    f32 = jnp.float32
    rowspec = pl.BlockSpec((TI, V), lambda i: (_tile_idx(i), 0))

    x2, s2t = pl.pallas_call(
        _mega,
        compiler_params=pltpu.CompilerParams(
            vmem_limit_bytes=63 * 1024 * 1024),
        grid=(STEPS,),
        in_specs=[_full((V, FIN)), rowspec, rowspec, _full((2, FIN)),
                  _full((FIN, H)), _full((FIN, H)), _full((FIN, H)),
                  _full((FIN, H)), _full((FIN, H)),
                  _full((H, H)), _full((H, H)), _full((H, H)),
                  _full((H, H)), _full((H, H))],
        out_specs=[_full((V, H)),
                   pl.BlockSpec((TI, H, V),
                                lambda i: (jnp.clip(i - P3, 0, NT - 1), 0, 0))],
        out_shape=[jax.ShapeDtypeStruct((V, H), f32),
                   jax.ShapeDtypeStruct((V, H, V), f32)],
        scratch_shapes=[
            pltpu.VMEM((V, H), f32),
            pltpu.VMEM((V, H), f32),
            pltpu.VMEM((8, H), f32),
            pltpu.VMEM((8, H), f32),
            pltpu.VMEM((V, H), f32),
            pltpu.VMEM((V, H), f32),
            pltpu.VMEM((V, H), f32),
            pltpu.VMEM((V, H, V), jnp.bfloat16),
            pltpu.VMEM((V, H), f32),
            pltpu.VMEM((V, H), f32),
            pltpu.VMEM((H, V), f32),
            pltpu.VMEM((H, V), f32),
            pltpu.VMEM((H, V), f32),
            pltpu.VMEM((H, V), f32),
            pltpu.VMEM((H, 8), f32),
            pltpu.VMEM((H, 8), f32),
            pltpu.VMEM((H, 8), f32),
            pltpu.VMEM((H, H), f32),
        ],
    )(x2d, Si, Ai, emb, Wa1, Wb1, Wc1, Wv1, Wu1, Wa2, Wb2, Wv2, Wu2, Wc2)

    s2 = jnp.transpose(s2t, (0, 2, 1))
    return (x2[None], s2[None])


# speedup vs baseline: 1.0433x; 1.0125x over previous
import functools

import jax
import jax.numpy as jnp
from jax.experimental import pallas as pl
from jax.experimental.pallas import tpu as pltpu

V, FIN, H = 512, 32, 64
TI = 32
NT = V // TI
PA, P2, PB, P3 = NT, NT + 1, 2 * NT + 1, 2 * NT + 2
STEPS = 3 * NT + 2
HIGH = jax.lax.Precision.HIGHEST


def _e1t(row0, sv, a1_scr, bc0t_scr, cm_scr):
    a1t = a1_scr[pl.ds(row0, TI), :]
    cdt = cm_scr[:, 0:1]
    return (a1t[:, :, None] + bc0t_scr[...][None, :, :]
            + sv[:, None, :] * cdt[None, :, :])


def _s1t(row0, sv, a1_scr, bc0t_scr, cm_scr, st1t_scr):
    e1t = _e1t(row0, sv, a1_scr, bc0t_scr, cm_scr)
    mu1t = st1t_scr[:, 0:1][None, :, :]
    inv1t = st1t_scr[:, 1:2][None, :, :]
    return jax.nn.relu((e1t - mu1t) * inv1t)


def _norm_relu(h):
    mu = jnp.mean(h, axis=0, keepdims=True)
    var = jnp.mean((h - mu) ** 2, axis=0, keepdims=True)
    return jax.nn.relu((h - mu) / jnp.sqrt(var + 1e-5))


def _mega(x_ref, s_ref, a_ref, emb_ref,
          wa1_ref, wb1_ref, wc1_ref, wv1_ref, wu1_ref,
          wa2_ref, wb2_ref, wv2_ref, wu2_ref, wc2_ref,
          x2_ref, s2_ref,
          num_scr, den_scr, raw1_scr, raw2_scr,
          x1_scr, a2_scr, u2_scr, e1s_scr,
          a1_scr, bc0_scr, bc0t_scr, v1t_scr, b2t_scr, v2t_scr,
          cm_scr, st1t_scr, st2t_scr, wc2t_scr):
    i = pl.program_id(0)

    @pl.when(i == 0)
    def _init():
        xf = x_ref[...]
        a1_scr[...] = jnp.dot(xf, wa1_ref[...], precision=HIGH)
        c = jnp.dot(emb_ref[...], wc1_ref[...], precision=HIGH)
        c0 = c[0:1, :]
        cd = c[1:2, :] - c0
        bc0 = jnp.dot(xf, wb1_ref[...], precision=HIGH) + c0
        bc0_scr[...] = bc0
        bc0t_scr[...] = jnp.transpose(bc0)
        v1t_scr[...] = jnp.transpose(
            jnp.dot(xf, wv1_ref[...], precision=HIGH))
        cm_scr[...] = jnp.concatenate(
            [jnp.transpose(cd), jnp.zeros((H, 7), jnp.float32)], axis=1)
        wc2t_scr[...] = jnp.transpose(wc2_ref[...])

    @pl.when(i < PA)
    def _phase1():
        li = i
        row0 = li * TI
        sv = s_ref[...].astype(jnp.float32)
        e1t = _e1t(row0, sv, a1_scr, bc0t_scr, cm_scr)
        e1s_scr[pl.ds(row0, TI), :, :] = e1t.astype(jnp.bfloat16)
        adj = jnp.minimum(a_ref[...] + s_ref[...], 1).astype(jnp.float32)
        g = 0.5 * jnp.tanh(0.5 * e1t) + 0.5
        ga = g * adj[:, None, :]
        num_scr[pl.ds(row0, TI), :] = jnp.sum(
            ga * v1t_scr[...][None, :, :], axis=2)
        den_scr[pl.ds(row0, TI), :] = jnp.sum(ga, axis=2)
        rt = jnp.sum(sv, axis=1, keepdims=True)
        qt = jnp.sum(sv, axis=0, keepdims=True)
        ca = jnp.sum(a1_scr[pl.ds(row0, TI), :] * rt, axis=0, keepdims=True)
        cb = jnp.transpose(
            jnp.sum(bc0t_scr[...] * qt, axis=1, keepdims=True))
        n1 = jnp.broadcast_to(jnp.sum(rt, axis=0, keepdims=True), (1, H))
        upd = jnp.concatenate(
            [ca, cb, n1, jnp.zeros((5, H), jnp.float32)], axis=0)

        @pl.when(li == 0)
        def _():
            raw1_scr[...] = upd

        @pl.when(li > 0)
        def _():
            raw1_scr[...] = raw1_scr[...] + upd

    @pl.when(i == PA)
    def _prep_a():
        xf = x_ref[...]
        ux1 = jnp.dot(xf, wu1_ref[...], precision=HIGH)
        x1 = _norm_relu(ux1 + num_scr[...] / (den_scr[...] + 1e-6))
        x1_scr[...] = x1
        a2_scr[...] = jnp.dot(x1, wa2_ref[...], precision=HIGH)
        b2t_scr[...] = jnp.transpose(
            jnp.dot(x1, wb2_ref[...], precision=HIGH))
        v2t_scr[...] = jnp.transpose(
            jnp.dot(x1, wv2_ref[...], precision=HIGH))
        u2_scr[...] = jnp.dot(x1, wu2_ref[...], precision=HIGH)
        a1 = a1_scr[...]
        bc0 = bc0_scr[...]
        cd = jnp.transpose(cm_scr[:, 0:1])
        sa = jnp.sum(a1, axis=0, keepdims=True)
        sa2 = jnp.sum(a1 * a1, axis=0, keepdims=True)
        sb = jnp.sum(bc0, axis=0, keepdims=True)
        sb2 = jnp.sum(bc0 * bc0, axis=0, keepdims=True)
        ca = raw1_scr[0:1, :]
        cb = raw1_scr[1:2, :]
        n1 = raw1_scr[2:3, :]
        fV = jnp.float32(V)
        n = fV * fV
        se = fV * sa + fV * sb + n1 * cd
        se2 = (fV * sa2 + fV * sb2 + 2.0 * sa * sb
               + 2.0 * cd * (ca + cb) + cd * cd * n1)
        mu1 = se / n
        var1 = se2 / n - mu1 * mu1
        inv1 = jax.lax.rsqrt(var1 + 1e-5)
        st1t_scr[...] = jnp.concatenate(
            [jnp.transpose(mu1), jnp.transpose(inv1),
             jnp.zeros((H, 6), jnp.float32)], axis=1)

    @pl.when((i > PA) & (i < PB))
    def _phase2():
        li = i - P2
        row0 = li * TI
        e1b = e1s_scr[pl.ds(row0, TI), :, :].astype(jnp.float32)
        mu1t = st1t_scr[:, 0:1][None, :, :]
        inv1t = st1t_scr[:, 1:2][None, :, :]
        s1t = jax.nn.relu((e1b - mu1t) * inv1t)
        e1s_scr[pl.ds(row0, TI), :, :] = s1t.astype(jnp.bfloat16)
        wc2t = jnp.broadcast_to(wc2t_scr[...][None, :, :], (TI, H, H))
        sct = jax.lax.dot_general(
            wc2t, s1t, (((2,), (1,)), ((0,), (0,))))
        a2t = a2_scr[pl.ds(row0, TI), :]
        e2t = a2t[:, :, None] + b2t_scr[...][None, :, :] + sct
        adj = jnp.minimum(a_ref[...] + s_ref[...], 1).astype(jnp.float32)
        g = 0.5 * jnp.tanh(0.5 * e2t) + 0.5
        ga = g * adj[:, None, :]
        num_scr[pl.ds(row0, TI), :] = jnp.sum(
            ga * v2t_scr[...][None, :, :], axis=2)
        den_scr[pl.ds(row0, TI), :] = jnp.sum(ga, axis=2)
        es = jnp.transpose(jnp.sum(e2t, axis=(0, 2))[:, None])
        eq = jnp.transpose(jnp.sum(e2t * e2t, axis=(0, 2))[:, None])
        upd = jnp.concatenate(
            [es, eq, jnp.zeros((6, H), jnp.float32)], axis=0)

        @pl.when(li == 0)
        def _():
            raw2_scr[...] = upd

        @pl.when(li > 0)
        def _():
            raw2_scr[...] = raw2_scr[...] + upd

    @pl.when(i == PB)
    def _prep_b():
        h = u2_scr[...] + num_scr[...] / (den_scr[...] + 1e-6)
        x2_ref[...] = jnp.transpose(x1_scr[...] + _norm_relu(h))
        n = jnp.float32(V) * jnp.float32(V)
        mu2 = raw2_scr[0:1, :] / n
        var2 = raw2_scr[1:2, :] / n - mu2 * mu2
        inv2 = jax.lax.rsqrt(var2 + 1e-5)
        st2t_scr[...] = jnp.concatenate(
            [jnp.transpose(mu2), jnp.transpose(inv2),
             jnp.zeros((H, 6), jnp.float32)], axis=1)

    @pl.when(i >= P3)
    def _phase3():
        li = i - P3
        row0 = li * TI
        s1t = e1s_scr[pl.ds(row0, TI), :, :].astype(jnp.float32)
        wc2t = jnp.broadcast_to(wc2t_scr[...][None, :, :], (TI, H, H))
        sct = jax.lax.dot_general(
            wc2t, s1t, (((2,), (1,)), ((0,), (0,))))
        a2t = a2_scr[pl.ds(row0, TI), :]
        e2t = a2t[:, :, None] + b2t_scr[...][None, :, :] + sct
        mu2t = st2t_scr[:, 0:1][None, :, :]
        inv2t = st2t_scr[:, 1:2][None, :, :]
        s2_ref[...] = s1t + jax.nn.relu((e2t - mu2t) * inv2t)


def _tile_idx(i):
    return jnp.where(i < P2, jnp.minimum(i, NT - 1),
                     jnp.minimum(i - P2, NT - 1))


def _full(shape):
    return pl.BlockSpec(shape, lambda i: tuple(0 for _ in shape))


@functools.partial(jax.jit, static_argnums=())
def kernel(x, A, S, emb, Wu1, Wv1, Wa1, Wb1, Wc1, Wu2, Wv2, Wa2, Wb2, Wc2):
    x2d = x[0]
    Si = S[0].astype(jnp.int32)
    Ai = A[0].astype(jnp.int32)

    f32 = jnp.float32
    rowspec = pl.BlockSpec((TI, V), lambda i: (_tile_idx(i), 0))

    x2, s2t = pl.pallas_call(
        _mega,
        compiler_params=pltpu.CompilerParams(
            vmem_limit_bytes=63 * 1024 * 1024),
        grid=(STEPS,),
        in_specs=[_full((V, FIN)), rowspec, rowspec, _full((2, FIN)),
                  _full((FIN, H)), _full((FIN, H)), _full((FIN, H)),
                  _full((FIN, H)), _full((FIN, H)),
                  _full((H, H)), _full((H, H)), _full((H, H)),
                  _full((H, H)), _full((H, H))],
        out_specs=[_full((H, V)),
                   pl.BlockSpec((TI, H, V),
                                lambda i: (jnp.clip(i - P3, 0, NT - 1), 0, 0))],
        out_shape=[jax.ShapeDtypeStruct((H, V), f32),
                   jax.ShapeDtypeStruct((V, H, V), f32)],
        scratch_shapes=[
            pltpu.VMEM((V, H), f32),
            pltpu.VMEM((V, H), f32),
            pltpu.VMEM((8, H), f32),
            pltpu.VMEM((8, H), f32),
            pltpu.VMEM((V, H), f32),
            pltpu.VMEM((V, H), f32),
            pltpu.VMEM((V, H), f32),
            pltpu.VMEM((V, H, V), jnp.bfloat16),
            pltpu.VMEM((V, H), f32),
            pltpu.VMEM((V, H), f32),
            pltpu.VMEM((H, V), f32),
            pltpu.VMEM((H, V), f32),
            pltpu.VMEM((H, V), f32),
            pltpu.VMEM((H, V), f32),
            pltpu.VMEM((H, 8), f32),
            pltpu.VMEM((H, 8), f32),
            pltpu.VMEM((H, 8), f32),
            pltpu.VMEM((H, H), f32),
        ],
    )(x2d, Si, Ai, emb, Wa1, Wb1, Wc1, Wv1, Wu1, Wa2, Wb2, Wv2, Wu2, Wc2)

    s2 = jnp.transpose(s2t, (0, 2, 1))
    return (jnp.transpose(x2)[None], s2[None])


# speedup vs baseline: 1.0577x; 1.0138x over previous
import functools

import jax
import jax.numpy as jnp
from jax.experimental import pallas as pl
from jax.experimental.pallas import tpu as pltpu

V, FIN, H = 512, 32, 64
TI = 32
NT = V // TI
PA, P2, PB, P3 = NT, NT + 1, 2 * NT + 1, 2 * NT + 2
STEPS = 3 * NT + 2
HIGH = jax.lax.Precision.HIGHEST


def _e1t(row0, sv, a1_scr, bc0t_scr, cm_scr):
    a1t = a1_scr[pl.ds(row0, TI), :]
    cdt = cm_scr[:, 0:1]
    return (a1t[:, :, None] + bc0t_scr[...][None, :, :]
            + sv[:, None, :] * cdt[None, :, :])


def _s1t(row0, sv, a1_scr, bc0t_scr, cm_scr, st1t_scr):
    e1t = _e1t(row0, sv, a1_scr, bc0t_scr, cm_scr)
    mu1t = st1t_scr[:, 0:1][None, :, :]
    inv1t = st1t_scr[:, 1:2][None, :, :]
    return jax.nn.relu((e1t - mu1t) * inv1t)


def _norm_relu(h):
    mu = jnp.mean(h, axis=0, keepdims=True)
    var = jnp.mean((h - mu) ** 2, axis=0, keepdims=True)
    return jax.nn.relu((h - mu) / jnp.sqrt(var + 1e-5))


def _mega(x_ref, s_ref, a_ref, emb_ref,
          wa1_ref, wb1_ref, wc1_ref, wv1_ref, wu1_ref,
          wa2_ref, wb2_ref, wv2_ref, wu2_ref, wc2_ref,
          x2_ref, s2_ref,
          num_scr, den_scr, raw1_scr, raw2_scr,
          x1_scr, a2_scr, u2_scr, e1s_scr,
          a1_scr, bc0_scr, bc0t_scr, v1t_scr, b2t_scr, v2t_scr,
          cm_scr, st1t_scr, st2t_scr, wc2t_scr):
    i = pl.program_id(0)

    def _xdot(w_ref):
        return jax.lax.dot_general(
            x_ref[...], w_ref[...], (((0,), (0,)), ((), ())), precision=HIGH)

    @pl.when(i == 0)
    def _init():
        a1_scr[...] = _xdot(wa1_ref)
        c = jnp.dot(emb_ref[...], wc1_ref[...], precision=HIGH)
        c0 = c[0:1, :]
        cd = c[1:2, :] - c0
        bc0 = _xdot(wb1_ref) + c0
        bc0_scr[...] = bc0
        bc0t_scr[...] = jnp.transpose(bc0)
        v1t_scr[...] = jnp.transpose(_xdot(wv1_ref))
        cm_scr[...] = jnp.concatenate(
            [jnp.transpose(cd), jnp.zeros((H, 7), jnp.float32)], axis=1)
        wc2t_scr[...] = jnp.transpose(wc2_ref[...])

    @pl.when(i < PA)
    def _phase1():
        li = i
        row0 = li * TI
        sv = s_ref[...].astype(jnp.float32)
        e1t = _e1t(row0, sv, a1_scr, bc0t_scr, cm_scr)
        e1s_scr[pl.ds(row0, TI), :, :] = e1t.astype(jnp.bfloat16)
        adj = jnp.minimum(a_ref[...] + s_ref[...], 1).astype(jnp.float32)
        g = 0.5 * jnp.tanh(0.5 * e1t) + 0.5
        ga = g * adj[:, None, :]
        num_scr[pl.ds(row0, TI), :] = jnp.sum(
            ga * v1t_scr[...][None, :, :], axis=2)
        den_scr[pl.ds(row0, TI), :] = jnp.sum(ga, axis=2)
        rt = jnp.sum(sv, axis=1, keepdims=True)
        qt = jnp.sum(sv, axis=0, keepdims=True)
        ca = jnp.sum(a1_scr[pl.ds(row0, TI), :] * rt, axis=0, keepdims=True)
        cb = jnp.transpose(
            jnp.sum(bc0t_scr[...] * qt, axis=1, keepdims=True))
        n1 = jnp.broadcast_to(jnp.sum(rt, axis=0, keepdims=True), (1, H))
        upd = jnp.concatenate(
            [ca, cb, n1, jnp.zeros((5, H), jnp.float32)], axis=0)

        @pl.when(li == 0)
        def _():
            raw1_scr[...] = upd

        @pl.when(li > 0)
        def _():
            raw1_scr[...] = raw1_scr[...] + upd

    @pl.when(i == PA)
    def _prep_a():
        ux1 = _xdot(wu1_ref)
        x1 = _norm_relu(ux1 + num_scr[...] / (den_scr[...] + 1e-6))
        x1_scr[...] = x1
        a2_scr[...] = jnp.dot(x1, wa2_ref[...], precision=HIGH)
        b2t_scr[...] = jnp.transpose(
            jnp.dot(x1, wb2_ref[...], precision=HIGH))
        v2t_scr[...] = jnp.transpose(
            jnp.dot(x1, wv2_ref[...], precision=HIGH))
        u2_scr[...] = jnp.dot(x1, wu2_ref[...], precision=HIGH)
        a1 = a1_scr[...]
        bc0 = bc0_scr[...]
        cd = jnp.transpose(cm_scr[:, 0:1])
        sa = jnp.sum(a1, axis=0, keepdims=True)
        sa2 = jnp.sum(a1 * a1, axis=0, keepdims=True)
        sb = jnp.sum(bc0, axis=0, keepdims=True)
        sb2 = jnp.sum(bc0 * bc0, axis=0, keepdims=True)
        ca = raw1_scr[0:1, :]
        cb = raw1_scr[1:2, :]
        n1 = raw1_scr[2:3, :]
        fV = jnp.float32(V)
        n = fV * fV
        se = fV * sa + fV * sb + n1 * cd
        se2 = (fV * sa2 + fV * sb2 + 2.0 * sa * sb
               + 2.0 * cd * (ca + cb) + cd * cd * n1)
        mu1 = se / n
        var1 = se2 / n - mu1 * mu1
        inv1 = jax.lax.rsqrt(var1 + 1e-5)
        st1t_scr[...] = jnp.concatenate(
            [jnp.transpose(mu1), jnp.transpose(inv1),
             jnp.zeros((H, 6), jnp.float32)], axis=1)

    @pl.when((i > PA) & (i < PB))
    def _phase2():
        li = i - P2
        row0 = li * TI
        e1b = e1s_scr[pl.ds(row0, TI), :, :].astype(jnp.float32)
        mu1t = st1t_scr[:, 0:1][None, :, :]
        inv1t = st1t_scr[:, 1:2][None, :, :]
        s1t = jax.nn.relu((e1b - mu1t) * inv1t)
        e1s_scr[pl.ds(row0, TI), :, :] = s1t.astype(jnp.bfloat16)
        wc2t = jnp.broadcast_to(wc2t_scr[...][None, :, :], (TI, H, H))
        sct = jax.lax.dot_general(
            wc2t, s1t, (((2,), (1,)), ((0,), (0,))))
        a2t = a2_scr[pl.ds(row0, TI), :]
        e2t = a2t[:, :, None] + b2t_scr[...][None, :, :] + sct
        adj = jnp.minimum(a_ref[...] + s_ref[...], 1).astype(jnp.float32)
        g = 0.5 * jnp.tanh(0.5 * e2t) + 0.5
        ga = g * adj[:, None, :]
        num_scr[pl.ds(row0, TI), :] = jnp.sum(
            ga * v2t_scr[...][None, :, :], axis=2)
        den_scr[pl.ds(row0, TI), :] = jnp.sum(ga, axis=2)
        es = jnp.transpose(jnp.sum(e2t, axis=(0, 2))[:, None])
        eq = jnp.transpose(jnp.sum(e2t * e2t, axis=(0, 2))[:, None])
        upd = jnp.concatenate(
            [es, eq, jnp.zeros((6, H), jnp.float32)], axis=0)

        @pl.when(li == 0)
        def _():
            raw2_scr[...] = upd

        @pl.when(li > 0)
        def _():
            raw2_scr[...] = raw2_scr[...] + upd

    @pl.when(i == PB)
    def _prep_b():
        h = u2_scr[...] + num_scr[...] / (den_scr[...] + 1e-6)
        x2_ref[...] = jnp.transpose(x1_scr[...] + _norm_relu(h))
        n = jnp.float32(V) * jnp.float32(V)
        mu2 = raw2_scr[0:1, :] / n
        var2 = raw2_scr[1:2, :] / n - mu2 * mu2
        inv2 = jax.lax.rsqrt(var2 + 1e-5)
        st2t_scr[...] = jnp.concatenate(
            [jnp.transpose(mu2), jnp.transpose(inv2),
             jnp.zeros((H, 6), jnp.float32)], axis=1)

    @pl.when(i >= P3)
    def _phase3():
        li = i - P3
        row0 = li * TI
        s1t = e1s_scr[pl.ds(row0, TI), :, :].astype(jnp.float32)
        wc2t = jnp.broadcast_to(wc2t_scr[...][None, :, :], (TI, H, H))
        sct = jax.lax.dot_general(
            wc2t, s1t, (((2,), (1,)), ((0,), (0,))))
        a2t = a2_scr[pl.ds(row0, TI), :]
        e2t = a2t[:, :, None] + b2t_scr[...][None, :, :] + sct
        mu2t = st2t_scr[:, 0:1][None, :, :]
        inv2t = st2t_scr[:, 1:2][None, :, :]
        s2_ref[...] = s1t + jax.nn.relu((e2t - mu2t) * inv2t)


def _tile_idx(i):
    return jnp.where(i < P2, jnp.minimum(i, NT - 1),
                     jnp.minimum(i - P2, NT - 1))


def _full(shape):
    return pl.BlockSpec(shape, lambda i: tuple(0 for _ in shape))


@functools.partial(jax.jit, static_argnums=())
def kernel(x, A, S, emb, Wu1, Wv1, Wa1, Wb1, Wc1, Wu2, Wv2, Wa2, Wb2, Wc2):
    x2d = jnp.transpose(x[0])
    Si = S[0].astype(jnp.int32)
    Ai = A[0].astype(jnp.int32)

    f32 = jnp.float32
    rowspec = pl.BlockSpec((TI, V), lambda i: (_tile_idx(i), 0))

    x2, s2t = pl.pallas_call(
        _mega,
        compiler_params=pltpu.CompilerParams(
            vmem_limit_bytes=63 * 1024 * 1024),
        grid=(STEPS,),
        in_specs=[_full((FIN, V)), rowspec, rowspec, _full((2, FIN)),
                  _full((FIN, H)), _full((FIN, H)), _full((FIN, H)),
                  _full((FIN, H)), _full((FIN, H)),
                  _full((H, H)), _full((H, H)), _full((H, H)),
                  _full((H, H)), _full((H, H))],
        out_specs=[_full((H, V)),
                   pl.BlockSpec((TI, H, V),
                                lambda i: (jnp.clip(i - P3, 0, NT - 1), 0, 0))],
        out_shape=[jax.ShapeDtypeStruct((H, V), f32),
                   jax.ShapeDtypeStruct((V, H, V), f32)],
        scratch_shapes=[
            pltpu.VMEM((V, H), f32),
            pltpu.VMEM((V, H), f32),
            pltpu.VMEM((8, H), f32),
            pltpu.VMEM((8, H), f32),
            pltpu.VMEM((V, H), f32),
            pltpu.VMEM((V, H), f32),
            pltpu.VMEM((V, H), f32),
            pltpu.VMEM((V, H, V), jnp.bfloat16),
            pltpu.VMEM((V, H), f32),
            pltpu.VMEM((V, H), f32),
            pltpu.VMEM((H, V), f32),
            pltpu.VMEM((H, V), f32),
            pltpu.VMEM((H, V), f32),
            pltpu.VMEM((H, V), f32),
            pltpu.VMEM((H, 8), f32),
            pltpu.VMEM((H, 8), f32),
            pltpu.VMEM((H, 8), f32),
            pltpu.VMEM((H, H), f32),
        ],
    )(x2d, Si, Ai, emb, Wa1, Wb1, Wc1, Wv1, Wu1, Wa2, Wb2, Wv2, Wu2, Wc2)

    s2 = jnp.transpose(s2t, (0, 2, 1))
    return (jnp.transpose(x2)[None], s2[None])
